# Initial kernel scaffold; baseline (speedup 1.0000x reference)
import jax
import jax.numpy as jnp
from jax.experimental import pallas as pl


def kernel(x, edge_index, W_in, b_in, W_mid, b_mid, W_out, b_out):
    raise NotImplementedError("write your pallas kernel here")


# speedup vs baseline: 6.4701x; 6.4701x over previous
import functools

import jax
import jax.numpy as jnp
from jax import lax
from jax.experimental import pallas as pl
from jax.experimental.pallas import tpu as pltpu
from jax.experimental.pallas import tpu_sc as plsc

N = 10000
NPAD = 10240
DH = 256
HALF = 128
E = 320000
N_TILES = 16
CHUNK = 128
CHUNKS_PER_TILE = 157
EPAD = N_TILES * CHUNKS_PER_TILE * CHUNK
ROWS_PER_TILE = NPAD // N_TILES
BLK = 256
NBLK = NPAD // BLK


def _make_sc_agg(w: int):
    mesh = plsc.VectorSubcoreMesh(core_axis_name="c", subcore_axis_name="s")

    @functools.partial(
        pl.kernel,
        mesh=mesh,
        out_type=jax.ShapeDtypeStruct((2 * NPAD, w), jnp.float32),
        scratch_types=[
            pltpu.VMEM((CHUNK,), jnp.int32),
            pltpu.VMEM((CHUNK,), jnp.int32),
            pltpu.VMEM((CHUNK, w), jnp.float32),
            pltpu.VMEM_SHARED((NPAD, w), jnp.float32),
            pltpu.SemaphoreType.DMA,
        ],
    )
    def agg(table_hbm, src2_hbm, dst_hbm, zeros_hbm, out_hbm,
            src_v, dst_v, rows_v, acc_sh, sem):
        c = lax.axis_index("c")
        s = lax.axis_index("s")

        pltpu.sync_copy(zeros_hbm.at[pl.ds(s * ROWS_PER_TILE, ROWS_PER_TILE)],
                        acc_sh.at[pl.ds(s * ROWS_PER_TILE, ROWS_PER_TILE)])
        plsc.subcore_barrier()

        def body(i, carry):
            base = (s * CHUNKS_PER_TILE + i) * CHUNK
            pltpu.sync_copy(src2_hbm.at[pl.ds(c * EPAD + base, CHUNK)], src_v)
            pltpu.sync_copy(dst_hbm.at[pl.ds(base, CHUNK)], dst_v)
            pltpu.async_copy(table_hbm.at[src_v], rows_v, sem).wait()
            pltpu.sync_copy(rows_v, acc_sh.at[dst_v], add=True)
            return carry

        lax.fori_loop(0, CHUNKS_PER_TILE, body, 0)
        plsc.subcore_barrier()

        pltpu.sync_copy(
            acc_sh.at[pl.ds(s * ROWS_PER_TILE, ROWS_PER_TILE)],
            out_hbm.at[pl.ds(c * NPAD + s * ROWS_PER_TILE, ROWS_PER_TILE)])

    return agg


_sc_agg_main = _make_sc_agg(HALF)


def _make_sc_deg():
    mesh = plsc.VectorSubcoreMesh(core_axis_name="c", subcore_axis_name="s")

    @functools.partial(
        pl.kernel,
        mesh=mesh,
        out_type=jax.ShapeDtypeStruct((2 * NPAD, HALF), jnp.float32),
        scratch_types=[
            pltpu.VMEM((CHUNK,), jnp.int32),
            pltpu.VMEM((CHUNK, HALF), jnp.float32),
            pltpu.VMEM_SHARED((NPAD, HALF), jnp.float32),
        ],
    )
    def deg(ones_hbm, dst_hbm, zeros_hbm, out_hbm, dst_v, ones_v, acc_sh):
        c = lax.axis_index("c")
        s = lax.axis_index("s")
        pltpu.sync_copy(zeros_hbm.at[pl.ds(s * ROWS_PER_TILE, ROWS_PER_TILE)],
                        acc_sh.at[pl.ds(s * ROWS_PER_TILE, ROWS_PER_TILE)])
        pltpu.sync_copy(ones_hbm, ones_v)
        plsc.subcore_barrier()

        def body(i, carry):
            base = (s * CHUNKS_PER_TILE + i) * CHUNK
            pltpu.sync_copy(dst_hbm.at[pl.ds(base, CHUNK)], dst_v)
            pltpu.sync_copy(ones_v, acc_sh.at[dst_v], add=True)
            return carry

        lax.fori_loop(0, CHUNKS_PER_TILE, body, 0)
        plsc.subcore_barrier()
        pltpu.sync_copy(
            acc_sh.at[pl.ds(s * ROWS_PER_TILE, ROWS_PER_TILE)],
            out_hbm.at[pl.ds(c * NPAD + s * ROWS_PER_TILE, ROWS_PER_TILE)])

    return deg


_sc_deg = _make_sc_deg()


def _dinv_body(deg_ref, dinv_ref):
    i = pl.program_id(0)
    deg = deg_ref[:, 0:1]
    row = i * BLK + lax.broadcasted_iota(jnp.int32, (BLK, 1), 0)
    dinv = jnp.where(row < N, lax.rsqrt(deg + 1.0), 0.0)
    dinv_ref[...] = jnp.broadcast_to(dinv, (BLK, HALF))


def _dinv_pass(deg):
    return pl.pallas_call(
        _dinv_body,
        grid=(NBLK,),
        in_specs=[pl.BlockSpec((BLK, HALF), lambda i: (i, 0))],
        out_specs=pl.BlockSpec((BLK, HALF), lambda i: (i, 0)),
        out_shape=jax.ShapeDtypeStruct((NPAD, HALF), jnp.float32),
    )(deg)


def _first_body(x_ref, w_ref, dinv_ref, y_ref):
    xw = jnp.dot(x_ref[...], w_ref[...], preferred_element_type=jnp.float32)
    y_ref[...] = xw * dinv_ref[:, 0:1]


def _first_pass(x, w_in, dinv):
    return pl.pallas_call(
        _first_body,
        grid=(NBLK, 2),
        in_specs=[
            pl.BlockSpec((BLK, HALF), lambda i, j: (i, 0)),
            pl.BlockSpec((HALF, HALF), lambda i, j: (0, j)),
            pl.BlockSpec((BLK, HALF), lambda i, j: (i, 0)),
        ],
        out_specs=pl.BlockSpec((BLK, HALF), lambda i, j: (j * NBLK + i, 0)),
        out_shape=jax.ShapeDtypeStruct((2 * NPAD, HALF), jnp.float32),
    )(x, w_in, dinv)


def _combine_body(aggl_ref, aggr_ref, yl_ref, yr_ref, dinv_ref, b_ref, w_ref,
                  y_next_ref):
    dinv = dinv_ref[:, 0:1]
    agg = jnp.concatenate([aggl_ref[...], aggr_ref[...]], axis=1)
    y = jnp.concatenate([yl_ref[...], yr_ref[...]], axis=1)
    h = jnp.maximum((agg + y) * dinv + b_ref[0:1, :], 0.0)
    yn = jnp.dot(h, w_ref[...], preferred_element_type=jnp.float32)
    y_next_ref[...] = yn * dinv


def _combine_pass(agg, y, dinv, b2, w_next):
    return pl.pallas_call(
        _combine_body,
        grid=(NBLK, 2),
        in_specs=[
            pl.BlockSpec((BLK, HALF), lambda i, j: (i, 0)),
            pl.BlockSpec((BLK, HALF), lambda i, j: (NBLK + i, 0)),
            pl.BlockSpec((BLK, HALF), lambda i, j: (i, 0)),
            pl.BlockSpec((BLK, HALF), lambda i, j: (NBLK + i, 0)),
            pl.BlockSpec((BLK, HALF), lambda i, j: (i, 0)),
            pl.BlockSpec((8, DH), lambda i, j: (0, 0)),
            pl.BlockSpec((DH, HALF), lambda i, j: (0, j)),
        ],
        out_specs=pl.BlockSpec((BLK, HALF), lambda i, j: (j * NBLK + i, 0)),
        out_shape=jax.ShapeDtypeStruct((2 * NPAD, HALF), jnp.float32),
    )(agg, agg, y, y, dinv, b2, w_next)


def _final_body(aggl_ref, yl_ref, dinv_ref, b_ref, out_ref):
    out_ref[...] = ((aggl_ref[...] + yl_ref[...]) * dinv_ref[:, 0:1]
                    + b_ref[0:1, :])


def _final_pass(agg, y, dinv, b2):
    return pl.pallas_call(
        _final_body,
        grid=(NBLK,),
        in_specs=[
            pl.BlockSpec((BLK, HALF), lambda i: (i, 0)),
            pl.BlockSpec((BLK, HALF), lambda i: (i, 0)),
            pl.BlockSpec((BLK, HALF), lambda i: (i, 0)),
            pl.BlockSpec((8, HALF), lambda i: (0, 0)),
        ],
        out_specs=pl.BlockSpec((BLK, HALF), lambda i: (i, 0)),
        out_shape=jax.ShapeDtypeStruct((NPAD, HALF), jnp.float32),
    )(agg, y, dinv, b2)


def kernel(x, edge_index, W_in, b_in, W_mid, b_mid, W_out, b_out):
    f32 = jnp.float32
    src = edge_index[0]
    dst = edge_index[1]

    npad_e = EPAD - E
    pad_idx = (N + (jnp.arange(npad_e, dtype=jnp.int32) % (NPAD - N))).astype(src.dtype)
    src_p = jnp.concatenate([src, pad_idx])
    dst_p = jnp.concatenate([dst, pad_idx])
    src2 = jnp.concatenate([src_p, src_p + NPAD])

    x_p = jnp.zeros((NPAD, HALF), f32).at[:N].set(x)
    zeros_main = jnp.zeros((NPAD, HALF), f32)
    ones_chunk = jnp.ones((CHUNK, HALF), f32)

    W_out_p = jnp.zeros((DH, DH), f32).at[:, :HALF].set(W_out)
    b2_in = jnp.broadcast_to(b_in[None, :], (8, DH))
    b2_mid = jnp.broadcast_to(b_mid[:, None, :], (18, 8, DH))
    b2_out = jnp.broadcast_to(b_out[None, :], (8, HALF))

    deg_stack = _sc_deg(ones_chunk, dst_p, zeros_main)
    dinv = _dinv_pass(deg_stack[:NPAD])

    y = _first_pass(x_p, W_in, dinv)

    w_next = [W_mid[i] for i in range(18)] + [W_out_p]
    b_cur = [b2_in] + [b2_mid[i] for i in range(18)]
    for l in range(19):
        agg = _sc_agg_main(y, src2, dst_p, zeros_main)
        y = _combine_pass(agg, y, dinv, b_cur[l], w_next[l])

    agg = _sc_agg_main(y, src2, dst_p, zeros_main)
    out = _final_pass(agg[:NPAD], y[:NPAD], dinv, b2_out)
    return out[:N]


# speedup vs baseline: 13.1427x; 2.0313x over previous
import functools

import jax
import jax.numpy as jnp
from jax import lax
from jax.experimental import pallas as pl
from jax.experimental.pallas import tpu as pltpu
from jax.experimental.pallas import tpu_sc as plsc

N = 10000
NPAD = 10240
DH = 256
HALF = 128
E = 320000
N_TILES = 16
CHUNK = 128
CHUNKS_PER_TILE = 160
EPAD = N_TILES * CHUNKS_PER_TILE * CHUNK
IDX_ROWS = EPAD // CHUNK
ROWS_PER_TILE = NPAD // N_TILES
BLK = 256
NBLK = NPAD // BLK


def _make_sc_agg(w: int):
    mesh = plsc.VectorSubcoreMesh(core_axis_name="c", subcore_axis_name="s")

    mesh = plsc.VectorSubcoreMesh(core_axis_name="c", subcore_axis_name="s")
    idx_t = pltpu.VMEM((1, 1, CHUNK), jnp.int32)

    @functools.partial(
        pl.kernel,
        mesh=mesh,
        out_type=jax.ShapeDtypeStruct((2 * NPAD, w), jnp.float32),
        scratch_types=(
            [idx_t] * 4 + [idx_t] * 4
            + [pltpu.VMEM((CHUNK, w), jnp.float32)] * 2
            + [pltpu.VMEM_SHARED((NPAD, w), jnp.float32)]
            + [pltpu.SemaphoreType.DMA] * 10
        ),
    )
    def agg(table_hbm, src2_hbm, dst2_hbm, zeros_hbm, out_hbm,
            sb0, sb1, sb2, sb3, db0, db1, db2, db3, r0, r1, acc_sh,
            is0, is1, is2, is3, id0, id1, id2, id3, gs0, gs1):
        c = lax.axis_index("c")
        s = lax.axis_index("s")
        srcb = (sb0, sb1, sb2, sb3)
        dstb = (db0, db1, db2, db3)
        isem = (is0, is1, is2, is3)
        dsem = (id0, id1, id2, id3)
        rows = (r0, r1)
        gsem = (gs0, gs1)
        srow = c * IDX_ROWS + s * CHUNKS_PER_TILE
        drow = s * CHUNKS_PER_TILE

        def idx_load(j, slot):
            pltpu.async_copy(src2_hbm.at[pl.ds(srow + j, 1)], srcb[slot], isem[slot])
            pltpu.async_copy(dst2_hbm.at[pl.ds(drow + j, 1)], dstb[slot], dsem[slot])

        def idx_wait_src(slot):
            pltpu.make_async_copy(src2_hbm.at[pl.ds(srow, 1)], srcb[slot], isem[slot]).wait()

        def idx_wait_dst(slot):
            pltpu.make_async_copy(dst2_hbm.at[pl.ds(drow, 1)], dstb[slot], dsem[slot]).wait()

        def gather(slot, p):
            pltpu.async_copy(table_hbm.at[srcb[slot].at[0].at[0]], rows[p], gsem[p])

        def gather_wait(p):
            pltpu.make_async_copy(table_hbm.at[srcb[0].at[0].at[0]], rows[p], gsem[p]).wait()

        idx_load(0, 0)
        idx_load(1, 1)
        pltpu.sync_copy(zeros_hbm.at[pl.ds(s * ROWS_PER_TILE, ROWS_PER_TILE)],
                        acc_sh.at[pl.ds(s * ROWS_PER_TILE, ROWS_PER_TILE)])
        plsc.subcore_barrier()
        idx_wait_src(0)
        gather(0, 0)

        def body(m, carry):
            for q in range(4):
                i = 4 * m + q
                idx_load(i + 2, (q + 2) % 4)
                idx_wait_src((q + 1) % 4)
                gather((q + 1) % 4, (q + 1) % 2)
                gather_wait(q % 2)
                idx_wait_dst(q)
                pltpu.sync_copy(rows[q % 2], acc_sh.at[dstb[q].at[0].at[0]],
                                add=True)
            return carry

        lax.fori_loop(0, CHUNKS_PER_TILE // 4, body, 0)
        gather_wait(0)
        idx_wait_src(1)
        idx_wait_dst(0)
        idx_wait_dst(1)
        plsc.subcore_barrier()

        pltpu.sync_copy(
            acc_sh.at[pl.ds(s * ROWS_PER_TILE, ROWS_PER_TILE)],
            out_hbm.at[pl.ds(c * NPAD + s * ROWS_PER_TILE, ROWS_PER_TILE)])

    return agg


_sc_agg_main = _make_sc_agg(HALF)


def _make_sc_deg():
    mesh = plsc.VectorSubcoreMesh(core_axis_name="c", subcore_axis_name="s")

    @functools.partial(
        pl.kernel,
        mesh=mesh,
        out_type=jax.ShapeDtypeStruct((2 * NPAD, HALF), jnp.float32),
        scratch_types=[
            pltpu.VMEM((CHUNKS_PER_TILE, 1, CHUNK), jnp.int32),
            pltpu.VMEM((CHUNK, HALF), jnp.float32),
            pltpu.VMEM_SHARED((NPAD, HALF), jnp.float32),
        ],
    )
    def deg(ones_hbm, dst2_hbm, zeros_hbm, out_hbm, dst_v, ones_v, acc_sh):
        c = lax.axis_index("c")
        s = lax.axis_index("s")
        pltpu.sync_copy(zeros_hbm.at[pl.ds(s * ROWS_PER_TILE, ROWS_PER_TILE)],
                        acc_sh.at[pl.ds(s * ROWS_PER_TILE, ROWS_PER_TILE)])
        pltpu.sync_copy(ones_hbm, ones_v)
        pltpu.sync_copy(dst2_hbm.at[pl.ds(s * CHUNKS_PER_TILE,
                                          CHUNKS_PER_TILE)], dst_v)
        plsc.subcore_barrier()

        def body(i, carry):
            pltpu.sync_copy(ones_v, acc_sh.at[dst_v.at[pl.ds(i, 1)].at[0].at[0]], add=True)
            return carry

        lax.fori_loop(0, CHUNKS_PER_TILE, body, 0)
        plsc.subcore_barrier()
        pltpu.sync_copy(
            acc_sh.at[pl.ds(s * ROWS_PER_TILE, ROWS_PER_TILE)],
            out_hbm.at[pl.ds(c * NPAD + s * ROWS_PER_TILE, ROWS_PER_TILE)])

    return deg


_sc_deg = _make_sc_deg()


def _dinv_body(deg_ref, dinv_ref):
    i = pl.program_id(0)
    deg = deg_ref[:, 0:1]
    row = i * BLK + lax.broadcasted_iota(jnp.int32, (BLK, 1), 0)
    dinv = jnp.where(row < N, lax.rsqrt(deg + 1.0), 0.0)
    dinv_ref[...] = jnp.broadcast_to(dinv, (BLK, HALF))


def _dinv_pass(deg):
    return pl.pallas_call(
        _dinv_body,
        grid=(NBLK,),
        in_specs=[pl.BlockSpec((BLK, HALF), lambda i: (i, 0))],
        out_specs=pl.BlockSpec((BLK, HALF), lambda i: (i, 0)),
        out_shape=jax.ShapeDtypeStruct((NPAD, HALF), jnp.float32),
    )(deg)


def _first_body(x_ref, w_ref, dinv_ref, y_ref):
    xw = jnp.dot(x_ref[...], w_ref[...], preferred_element_type=jnp.float32)
    y_ref[...] = xw * dinv_ref[:, 0:1]


def _first_pass(x, w_in, dinv):
    return pl.pallas_call(
        _first_body,
        grid=(NBLK, 2),
        in_specs=[
            pl.BlockSpec((BLK, HALF), lambda i, j: (i, 0)),
            pl.BlockSpec((HALF, HALF), lambda i, j: (0, j)),
            pl.BlockSpec((BLK, HALF), lambda i, j: (i, 0)),
        ],
        out_specs=pl.BlockSpec((BLK, HALF), lambda i, j: (j * NBLK + i, 0)),
        out_shape=jax.ShapeDtypeStruct((2 * NPAD, HALF), jnp.float32),
    )(x, w_in, dinv)


def _combine_body(aggl_ref, aggr_ref, yl_ref, yr_ref, dinv_ref, b_ref, w_ref,
                  y_next_ref):
    dinv = dinv_ref[:, 0:1]
    agg = jnp.concatenate([aggl_ref[...], aggr_ref[...]], axis=1)
    y = jnp.concatenate([yl_ref[...], yr_ref[...]], axis=1)
    h = jnp.maximum((agg + y) * dinv + b_ref[0:1, :], 0.0)
    yn = jnp.dot(h, w_ref[...], preferred_element_type=jnp.float32)
    y_next_ref[...] = yn * dinv


def _combine_pass(agg, y, dinv, b2, w_next):
    return pl.pallas_call(
        _combine_body,
        grid=(NBLK, 2),
        in_specs=[
            pl.BlockSpec((BLK, HALF), lambda i, j: (i, 0)),
            pl.BlockSpec((BLK, HALF), lambda i, j: (NBLK + i, 0)),
            pl.BlockSpec((BLK, HALF), lambda i, j: (i, 0)),
            pl.BlockSpec((BLK, HALF), lambda i, j: (NBLK + i, 0)),
            pl.BlockSpec((BLK, HALF), lambda i, j: (i, 0)),
            pl.BlockSpec((8, DH), lambda i, j: (0, 0)),
            pl.BlockSpec((DH, HALF), lambda i, j: (0, j)),
        ],
        out_specs=pl.BlockSpec((BLK, HALF), lambda i, j: (j * NBLK + i, 0)),
        out_shape=jax.ShapeDtypeStruct((2 * NPAD, HALF), jnp.float32),
    )(agg, agg, y, y, dinv, b2, w_next)


def _final_body(aggl_ref, yl_ref, dinv_ref, b_ref, out_ref):
    out_ref[...] = ((aggl_ref[...] + yl_ref[...]) * dinv_ref[:, 0:1]
                    + b_ref[0:1, :])


def _final_pass(agg, y, dinv, b2):
    return pl.pallas_call(
        _final_body,
        grid=(NBLK,),
        in_specs=[
            pl.BlockSpec((BLK, HALF), lambda i: (i, 0)),
            pl.BlockSpec((BLK, HALF), lambda i: (i, 0)),
            pl.BlockSpec((BLK, HALF), lambda i: (i, 0)),
            pl.BlockSpec((8, HALF), lambda i: (0, 0)),
        ],
        out_specs=pl.BlockSpec((BLK, HALF), lambda i: (i, 0)),
        out_shape=jax.ShapeDtypeStruct((NPAD, HALF), jnp.float32),
    )(agg, y, dinv, b2)


def kernel(x, edge_index, W_in, b_in, W_mid, b_mid, W_out, b_out):
    f32 = jnp.float32
    src = edge_index[0]
    dst = edge_index[1]

    npad_e = EPAD - E
    pad_idx = (N + (jnp.arange(npad_e, dtype=jnp.int32) % (NPAD - N))).astype(src.dtype)
    pad_row = (N + (jnp.arange(CHUNK, dtype=jnp.int32) % (NPAD - N))).astype(src.dtype)
    src_p = jnp.concatenate([src, pad_idx])
    dst_p = jnp.concatenate([dst, pad_idx])
    src2 = jnp.concatenate([src_p, src_p + NPAD, pad_row, pad_row])
    src2_2d = src2.reshape(2 * IDX_ROWS + 2, 1, CHUNK)
    dst_2d = jnp.concatenate([dst_p, pad_row, pad_row]).reshape(IDX_ROWS + 2, 1, CHUNK)

    x_p = jnp.zeros((NPAD, HALF), f32).at[:N].set(x)
    zeros_main = jnp.zeros((NPAD, HALF), f32)
    ones_chunk = jnp.ones((CHUNK, HALF), f32)

    W_out_p = jnp.zeros((DH, DH), f32).at[:, :HALF].set(W_out)
    b2_in = jnp.broadcast_to(b_in[None, :], (8, DH))
    b2_mid = jnp.broadcast_to(b_mid[:, None, :], (18, 8, DH))
    b2_out = jnp.broadcast_to(b_out[None, :], (8, HALF))

    deg_stack = _sc_deg(ones_chunk, dst_2d, zeros_main)
    dinv = _dinv_pass(deg_stack[:NPAD])

    y = _first_pass(x_p, W_in, dinv)

    w_next = [W_mid[i] for i in range(18)] + [W_out_p]
    b_cur = [b2_in] + [b2_mid[i] for i in range(18)]
    for l in range(19):
        agg = _sc_agg_main(y, src2_2d, dst_2d, zeros_main)
        y = _combine_pass(agg, y, dinv, b_cur[l], w_next[l])

    agg = _sc_agg_main(y, src2_2d, dst_2d, zeros_main)
    out = _final_pass(agg[:NPAD], y[:NPAD], dinv, b2_out)
    return out[:N]
